# Initial kernel scaffold; baseline (speedup 1.0000x reference)
import jax
import jax.numpy as jnp
from jax.experimental import pallas as pl


def kernel(h, edge_idx, W):
    raise NotImplementedError("write your pallas kernel here")


# speedup vs baseline: 15.8199x; 15.8199x over previous
import functools

import jax
import jax.numpy as jnp
from jax import lax
from jax.experimental import pallas as pl
from jax.experimental.pallas import tpu as pltpu
from jax.experimental.pallas import tpu_sc as plsc

N_NODE = 10000
N_EDGE = 160000
D_IN = 256
D_OUT = 64
N_HEAD = 4

HALF = 128
NSUB = 16
EDGES_PER_SUB = N_EDGE // NSUB
CHUNK = 80
NCHUNK = EDGES_PER_SUB // CHUNK
NSEG = 5
SEGCH = NCHUNK // NSEG
ROWS_PER_SUB = 624
ZROWS = 16
DEGR = 80


def _sc_aggregate(h01, rows_r, cols_r):
    mesh = plsc.VectorSubcoreMesh(core_axis_name="c", subcore_axis_name="s")

    @functools.partial(
        pl.kernel,
        out_type=(
            jax.ShapeDtypeStruct((2, N_NODE, HALF), jnp.float32),
            jax.ShapeDtypeStruct((2, DEGR, 128), jnp.float32),
        ),
        mesh=mesh,
        compiler_params=pltpu.CompilerParams(needs_layout_passes=False),
        scratch_types=[
            pltpu.VMEM((SEGCH, CHUNK), jnp.int32),
            pltpu.VMEM((SEGCH, CHUNK), jnp.int32),
            pltpu.VMEM((CHUNK,), jnp.int32),
            pltpu.VMEM((CHUNK,), jnp.int32),
            pltpu.VMEM((CHUNK, HALF), jnp.float32),
            pltpu.VMEM((ZROWS, HALF), jnp.float32),
            pltpu.VMEM((DEGR, 128), jnp.float32),
            pltpu.VMEM((DEGR,), jnp.int32),
            pltpu.VMEM_SHARED((N_NODE, HALF), jnp.float32),
            pltpu.VMEM_SHARED((DEGR, 128), jnp.float32),
            pltpu.SemaphoreType.DMA,
        ],
    )
    def k(h01_hbm, rows_hbm, cols_hbm, out_hbm, deg_hbm,
          rowbuf, colbuf, colidx, rowidx, gbuf, zbuf, degbuf, degids,
          acc, degacc, sem):
        c = lax.axis_index("c")
        s = lax.axis_index("s")

        zero16 = jnp.zeros((16,), jnp.float32)
        for r in range(ZROWS):
            for kk in range(HALF // 16):
                zbuf[r, pl.ds(kk * 16, 16)] = zero16

        def dz(i, carry):
            for kk in range(128 // 16):
                degbuf[i, pl.ds(kk * 16, 16)] = zero16
            return carry

        lax.fori_loop(0, DEGR, dz, 0)
        for kk in range(DEGR // 16):
            degids[pl.ds(kk * 16, 16)] = (
                lax.iota(jnp.int32, 16) + (kk * 16))

        def zbody(i, carry):
            pltpu.sync_copy(
                zbuf, acc.at[pl.ds(s * ROWS_PER_SUB + i * ZROWS, ZROWS), :])
            return carry

        lax.fori_loop(0, ROWS_PER_SUB // ZROWS, zbody, 0)

        @pl.when(s == NSUB - 1)
        def _():
            pltpu.sync_copy(zbuf, acc.at[pl.ds(NSUB * ROWS_PER_SUB, ZROWS), :])

        @pl.when(s == 0)
        def _():
            for i in range(DEGR // ZROWS):
                pltpu.sync_copy(zbuf, degacc.at[pl.ds(i * ZROWS, ZROWS), :])

        plsc.subcore_barrier()

        cbase = c * N_NODE

        def seg_body(g, carry):
            pltpu.sync_copy(rows_hbm.at[s, g], rowbuf)
            pltpu.sync_copy(cols_hbm.at[s, g], colbuf)

            def dbody(j, dcarry):
                for kk in range(CHUNK // 16):
                    v = rowbuf[j, pl.ds(kk * 16, 16)]
                    cnt, last = plsc.scan_count(v)
                    plsc.addupdate_scatter(
                        degbuf, [lax.shift_right_logical(v, 7),
                                 lax.bitwise_and(v, 127)],
                        cnt.astype(jnp.float32), mask=last)
                return dcarry

            lax.fori_loop(c * 12, 12 + c * 13, dbody, 0)

            def body(j, bcarry):
                for kk in range(CHUNK // 16):
                    sl = pl.ds(kk * 16, 16)
                    colidx[sl] = colbuf[j, sl] + cbase
                    rowidx[sl] = rowbuf[j, sl]
                pltpu.async_copy(h01_hbm.at[colidx], gbuf, sem).wait()
                pltpu.sync_copy(gbuf, acc.at[rowidx], add=True)
                return bcarry

            lax.fori_loop(0, SEGCH, body, 0)
            return carry

        lax.fori_loop(0, NSEG, seg_body, 0)

        pltpu.sync_copy(degbuf, degacc.at[degids], add=True)
        plsc.subcore_barrier()

        r0 = s * ROWS_PER_SUB
        pltpu.sync_copy(acc.at[pl.ds(r0, ROWS_PER_SUB), :],
                        out_hbm.at[c, pl.ds(r0, ROWS_PER_SUB), :])

        @pl.when(s == NSUB - 1)
        def _():
            t0 = NSUB * ROWS_PER_SUB
            pltpu.sync_copy(acc.at[pl.ds(t0, ZROWS), :],
                            out_hbm.at[c, pl.ds(t0, ZROWS), :])

        @pl.when(s == 0)
        def _():
            pltpu.sync_copy(degacc, deg_hbm.at[c])

    return k(h01, rows_r, cols_r)


def _tc_finish(a0, a1, deg_col, w0, w1):
    BR = 400
    grid = (N_NODE // BR,)

    def body(a0_ref, a1_ref, deg_ref, w0_ref, w1_ref, o_ref):
        x0 = a0_ref[...]
        x1 = a1_ref[...]
        r = 1.0 / jnp.maximum(deg_ref[...], 1.0)
        acc = jnp.dot(x0, w0_ref[...], preferred_element_type=jnp.float32)
        acc += jnp.dot(x1, w1_ref[...], preferred_element_type=jnp.float32)
        o_ref[...] = acc * r

    return pl.pallas_call(
        body,
        grid=grid,
        in_specs=[
            pl.BlockSpec((BR, HALF), lambda i: (i, 0)),
            pl.BlockSpec((BR, HALF), lambda i: (i, 0)),
            pl.BlockSpec((BR, 1), lambda i: (i, 0)),
            pl.BlockSpec((HALF, D_IN), lambda i: (0, 0)),
            pl.BlockSpec((HALF, D_IN), lambda i: (0, 0)),
        ],
        out_specs=pl.BlockSpec((BR, D_IN), lambda i: (i, 0)),
        out_shape=jax.ShapeDtypeStruct((N_NODE, N_HEAD * D_OUT), jnp.float32),
    )(a0, a1, deg_col, w0, w1)


def kernel(h, edge_idx, W):
    n, d_in = h.shape
    h01 = jnp.concatenate([h[:, :HALF], h[:, HALF:]], axis=0)

    rows_r = edge_idx[0].reshape(NSUB, NSEG, SEGCH, CHUNK)
    cols_r = edge_idx[1].reshape(NSUB, NSEG, SEGCH, CHUNK)

    agg01, deg01 = _sc_aggregate(h01, rows_r, cols_r)
    deg_col = (deg01[0] + deg01[1]).reshape(-1)[:n].reshape(n, 1)

    wcat = jnp.transpose(W, (1, 0, 2)).reshape(d_in, N_HEAD * D_OUT)

    return _tc_finish(agg01[0], agg01[1], deg_col, wcat[:HALF], wcat[HALF:])


# speedup vs baseline: 22.4840x; 1.4212x over previous
import functools

import jax
import jax.numpy as jnp
from jax import lax
from jax.experimental import pallas as pl
from jax.experimental.pallas import tpu as pltpu
from jax.experimental.pallas import tpu_sc as plsc

N_NODE = 10000
N_EDGE = 160000
D_IN = 256
D_OUT = 64
N_HEAD = 4

HALF = 128
NSUB = 16
EDGES_PER_SUB = N_EDGE // NSUB
CHUNK = 80
NCHUNK = EDGES_PER_SUB // CHUNK
NSEG = 5
SEGCH = NCHUNK // NSEG
ROWS_PER_SUB = 624
ZROWS = 16
DEGR = 80


def _sc_aggregate(h01, rows_r, cols_r):
    mesh = plsc.VectorSubcoreMesh(core_axis_name="c", subcore_axis_name="s")

    @functools.partial(
        pl.kernel,
        out_type=(
            jax.ShapeDtypeStruct((2, N_NODE, HALF), jnp.float32),
            jax.ShapeDtypeStruct((2, DEGR, 128), jnp.float32),
        ),
        mesh=mesh,
        compiler_params=pltpu.CompilerParams(needs_layout_passes=False),
        scratch_types=[
            pltpu.VMEM((SEGCH, CHUNK), jnp.int32),
            pltpu.VMEM((SEGCH, CHUNK), jnp.int32),
            pltpu.VMEM((CHUNK,), jnp.int32),
            pltpu.VMEM((CHUNK,), jnp.int32),
            pltpu.VMEM((CHUNK,), jnp.int32),
            pltpu.VMEM((CHUNK,), jnp.int32),
            pltpu.VMEM((CHUNK, HALF), jnp.float32),
            pltpu.VMEM((CHUNK, HALF), jnp.float32),
            pltpu.VMEM((ZROWS, HALF), jnp.float32),
            pltpu.VMEM((DEGR, 128), jnp.float32),
            pltpu.VMEM((DEGR,), jnp.int32),
            pltpu.VMEM_SHARED((N_NODE, HALF), jnp.float32),
            pltpu.VMEM_SHARED((DEGR, 128), jnp.float32),
            pltpu.SemaphoreType.DMA,
            pltpu.SemaphoreType.DMA,
            pltpu.SemaphoreType.DMA,
            pltpu.SemaphoreType.DMA,
        ],
    )
    def k(h01_hbm, rows_hbm, cols_hbm, out_hbm, deg_hbm,
          rowbuf, colbuf, colidx0, rowidx0, colidx1, rowidx1,
          gbuf0, gbuf1, zbuf, degbuf, degids,
          acc, degacc, semg0, semg1, sems0, sems1):
        c = lax.axis_index("c")
        s = lax.axis_index("s")

        zero16 = jnp.zeros((16,), jnp.float32)
        for r in range(ZROWS):
            for kk in range(HALF // 16):
                zbuf[r, pl.ds(kk * 16, 16)] = zero16

        def dz(i, carry):
            for kk in range(128 // 16):
                degbuf[i, pl.ds(kk * 16, 16)] = zero16
            return carry

        lax.fori_loop(0, DEGR, dz, 0)
        for kk in range(DEGR // 16):
            degids[pl.ds(kk * 16, 16)] = (
                lax.iota(jnp.int32, 16) + (kk * 16))

        def zbody(i, carry):
            pltpu.sync_copy(
                zbuf, acc.at[pl.ds(s * ROWS_PER_SUB + i * ZROWS, ZROWS), :])
            return carry

        lax.fori_loop(0, ROWS_PER_SUB // ZROWS, zbody, 0)

        @pl.when(s == NSUB - 1)
        def _():
            pltpu.sync_copy(zbuf, acc.at[pl.ds(NSUB * ROWS_PER_SUB, ZROWS), :])

        @pl.when(s == 0)
        def _():
            for i in range(DEGR // ZROWS):
                pltpu.sync_copy(zbuf, degacc.at[pl.ds(i * ZROWS, ZROWS), :])

        plsc.subcore_barrier()

        cbase = c * N_NODE

        def seg_body(g, carry):
            pltpu.sync_copy(rows_hbm.at[s, g], rowbuf)
            pltpu.sync_copy(cols_hbm.at[s, g], colbuf)

            def dbody(j, dcarry):
                for kk in range(CHUNK // 16):
                    v = rowbuf[j, pl.ds(kk * 16, 16)]
                    cnt, last = plsc.scan_count(v)
                    plsc.addupdate_scatter(
                        degbuf, [lax.shift_right_logical(v, 7),
                                 lax.bitwise_and(v, 127)],
                        cnt.astype(jnp.float32), mask=last)
                return dcarry

            lax.fori_loop(c * 12, 12 + c * 13, dbody, 0)

            def stage(j, cidx, ridx):
                for kk in range(CHUNK // 16):
                    sl = pl.ds(kk * 16, 16)
                    cidx[sl] = colbuf[j, sl] + cbase
                    ridx[sl] = rowbuf[j, sl]

            stage(0, colidx0, rowidx0)
            pltpu.async_copy(h01_hbm.at[colidx0], gbuf0, semg0)

            def pair(p, pcarry):
                j0 = 2 * p

                @pl.when(p > 0)
                def _():
                    pltpu.make_async_copy(gbuf1, acc.at[rowidx1],
                                          sems1).wait()

                stage(j0 + 1, colidx1, rowidx1)
                pltpu.async_copy(h01_hbm.at[colidx1], gbuf1, semg1)
                pltpu.make_async_copy(h01_hbm.at[colidx0], gbuf0,
                                      semg0).wait()
                pltpu.async_copy(gbuf0, acc.at[rowidx0], sems0, add=True)
                pltpu.make_async_copy(gbuf0, acc.at[rowidx0], sems0).wait()
                stage(j0 + 2, colidx0, rowidx0)
                pltpu.async_copy(h01_hbm.at[colidx0], gbuf0, semg0)
                pltpu.make_async_copy(h01_hbm.at[colidx1], gbuf1,
                                      semg1).wait()
                pltpu.async_copy(gbuf1, acc.at[rowidx1], sems1, add=True)
                return pcarry

            lax.fori_loop(0, (SEGCH - 1) // 2, pair, 0)
            pltpu.make_async_copy(gbuf1, acc.at[rowidx1], sems1).wait()
            pltpu.make_async_copy(h01_hbm.at[colidx0], gbuf0, semg0).wait()
            pltpu.sync_copy(gbuf0, acc.at[rowidx0], add=True)
            return carry

        lax.fori_loop(0, NSEG, seg_body, 0)

        pltpu.sync_copy(degbuf, degacc.at[degids], add=True)
        plsc.subcore_barrier()

        r0 = s * ROWS_PER_SUB
        pltpu.sync_copy(acc.at[pl.ds(r0, ROWS_PER_SUB), :],
                        out_hbm.at[c, pl.ds(r0, ROWS_PER_SUB), :])

        @pl.when(s == NSUB - 1)
        def _():
            t0 = NSUB * ROWS_PER_SUB
            pltpu.sync_copy(acc.at[pl.ds(t0, ZROWS), :],
                            out_hbm.at[c, pl.ds(t0, ZROWS), :])

        @pl.when(s == 0)
        def _():
            pltpu.sync_copy(degacc, deg_hbm.at[c])

    return k(h01, rows_r, cols_r)


def _tc_finish(a0, a1, deg_col, w0, w1):
    BR = 400
    grid = (N_NODE // BR,)

    def body(a0_ref, a1_ref, deg_ref, w0_ref, w1_ref, o_ref):
        x0 = a0_ref[...]
        x1 = a1_ref[...]
        r = 1.0 / jnp.maximum(deg_ref[...], 1.0)
        acc = jnp.dot(x0, w0_ref[...], preferred_element_type=jnp.float32)
        acc += jnp.dot(x1, w1_ref[...], preferred_element_type=jnp.float32)
        o_ref[...] = acc * r

    return pl.pallas_call(
        body,
        grid=grid,
        in_specs=[
            pl.BlockSpec((BR, HALF), lambda i: (i, 0)),
            pl.BlockSpec((BR, HALF), lambda i: (i, 0)),
            pl.BlockSpec((BR, 1), lambda i: (i, 0)),
            pl.BlockSpec((HALF, D_IN), lambda i: (0, 0)),
            pl.BlockSpec((HALF, D_IN), lambda i: (0, 0)),
        ],
        out_specs=pl.BlockSpec((BR, D_IN), lambda i: (i, 0)),
        out_shape=jax.ShapeDtypeStruct((N_NODE, N_HEAD * D_OUT), jnp.float32),
    )(a0, a1, deg_col, w0, w1)


def kernel(h, edge_idx, W):
    n, d_in = h.shape
    h01 = jnp.concatenate([h[:, :HALF], h[:, HALF:]], axis=0)

    rows_r = edge_idx[0].reshape(NSUB, NSEG, SEGCH, CHUNK)
    cols_r = edge_idx[1].reshape(NSUB, NSEG, SEGCH, CHUNK)

    agg01, deg01 = _sc_aggregate(h01, rows_r, cols_r)
    deg_col = (deg01[0] + deg01[1]).reshape(-1)[:n].reshape(n, 1)

    wcat = jnp.transpose(W, (1, 0, 2)).reshape(d_in, N_HEAD * D_OUT)

    return _tc_finish(agg01[0], agg01[1], deg_col, wcat[:HALF], wcat[HALF:])


# speedup vs baseline: 24.3408x; 1.0826x over previous
import functools

import jax
import jax.numpy as jnp
from jax import lax
from jax.experimental import pallas as pl
from jax.experimental.pallas import tpu as pltpu
from jax.experimental.pallas import tpu_sc as plsc

N_NODE = 10000
N_EDGE = 160000
D_IN = 256
D_OUT = 64
N_HEAD = 4

HALF = 128
NSUB = 16
EDGES_PER_SUB = N_EDGE // NSUB
CHUNK = 80
NCHUNK = EDGES_PER_SUB // CHUNK
NSEG = 5
SEGCH = NCHUNK // NSEG
ROWS_PER_SUB = 624
ZROWS = 48
DEGR = 80


def _sc_aggregate(h, rows_r, cols_r):
    mesh = plsc.VectorSubcoreMesh(core_axis_name="c", subcore_axis_name="s")

    @functools.partial(
        pl.kernel,
        out_type=(
            jax.ShapeDtypeStruct((2, N_NODE, HALF), jnp.float32),
            jax.ShapeDtypeStruct((2, DEGR, 128), jnp.float32),
        ),
        mesh=mesh,
        compiler_params=pltpu.CompilerParams(needs_layout_passes=False),
        scratch_types=[
            pltpu.VMEM((SEGCH, CHUNK), jnp.int32),
            pltpu.VMEM((SEGCH, CHUNK), jnp.int32),
            pltpu.VMEM((CHUNK,), jnp.int32),
            pltpu.VMEM((CHUNK,), jnp.int32),
            pltpu.VMEM((CHUNK, HALF), jnp.float32),
            pltpu.VMEM((CHUNK, HALF), jnp.float32),
            pltpu.VMEM((ZROWS, HALF), jnp.float32),
            pltpu.VMEM((DEGR, 128), jnp.float32),
            pltpu.VMEM((DEGR,), jnp.int32),
            pltpu.VMEM_SHARED((N_NODE, HALF), jnp.float32),
            pltpu.VMEM_SHARED((DEGR, 128), jnp.float32),
            pltpu.SemaphoreType.DMA,
            pltpu.SemaphoreType.DMA,
            pltpu.SemaphoreType.DMA,
            pltpu.SemaphoreType.DMA,
        ],
    )
    def k(h_hbm, rows_hbm, cols_hbm, out_hbm, deg_hbm,
          rowbuf, colbuf, rowidx0, rowidx1,
          gbuf0, gbuf1, zbuf, degbuf, degids,
          acc, degacc, semg0, semg1, sems0, sems1):
        c = lax.axis_index("c")
        s = lax.axis_index("s")

        zero16 = jnp.zeros((16,), jnp.float32)
        for r in range(ZROWS):
            for kk in range(HALF // 16):
                zbuf[r, pl.ds(kk * 16, 16)] = zero16

        def dz(i, carry):
            for kk in range(128 // 16):
                degbuf[i, pl.ds(kk * 16, 16)] = zero16
            return carry

        lax.fori_loop(0, DEGR, dz, 0)
        for kk in range(DEGR // 16):
            degids[pl.ds(kk * 16, 16)] = (
                lax.iota(jnp.int32, 16) + (kk * 16))

        def zbody(i, carry):
            pltpu.async_copy(
                zbuf, acc.at[pl.ds(s * ROWS_PER_SUB + i * ZROWS, ZROWS), :],
                semg0)
            return carry

        nz = ROWS_PER_SUB // ZROWS
        lax.fori_loop(0, nz, zbody, 0)

        @pl.when(s == NSUB - 1)
        def _():
            pltpu.async_copy(
                zbuf.at[pl.ds(0, 16)],
                acc.at[pl.ds(NSUB * ROWS_PER_SUB, 16), :], semg1)

        @pl.when(s == 0)
        def _():
            pltpu.async_copy(zbuf, degacc.at[pl.ds(0, ZROWS), :], semg1)
            pltpu.async_copy(zbuf.at[pl.ds(0, DEGR - ZROWS)],
                             degacc.at[pl.ds(ZROWS, DEGR - ZROWS), :], semg1)

        def zdrain(i, carry):
            pltpu.make_async_copy(
                zbuf, acc.at[pl.ds(s * ROWS_PER_SUB, ZROWS), :], semg0).wait()
            return carry

        lax.fori_loop(0, nz, zdrain, 0)

        @pl.when(s == NSUB - 1)
        def _():
            pltpu.make_async_copy(
                zbuf.at[pl.ds(0, 16)],
                acc.at[pl.ds(NSUB * ROWS_PER_SUB, 16), :], semg1).wait()

        @pl.when(s == 0)
        def _():
            pltpu.make_async_copy(
                zbuf, degacc.at[pl.ds(0, ZROWS), :], semg1).wait()
            pltpu.make_async_copy(
                zbuf.at[pl.ds(0, DEGR - ZROWS)],
                degacc.at[pl.ds(ZROWS, DEGR - ZROWS), :], semg1).wait()

        plsc.subcore_barrier()

        coff = pl.multiple_of(c * HALF, 128)

        def seg_body(g, carry):
            pltpu.sync_copy(rows_hbm.at[s, g], rowbuf)
            pltpu.sync_copy(cols_hbm.at[s, g], colbuf)

            def dbody(j, dcarry):
                for kk in range(CHUNK // 16):
                    v = rowbuf[j, pl.ds(kk * 16, 16)]
                    cnt, last = plsc.scan_count(v)
                    plsc.addupdate_scatter(
                        degbuf, [lax.shift_right_logical(v, 7),
                                 lax.bitwise_and(v, 127)],
                        cnt.astype(jnp.float32), mask=last)
                return dcarry

            lax.fori_loop(c * 12, 12 + c * 13, dbody, 0)

            def stage(j, ridx):
                for kk in range(CHUNK // 16):
                    sl = pl.ds(kk * 16, 16)
                    ridx[sl] = rowbuf[j, sl]

            def gather(j, gbuf, semg):
                pltpu.async_copy(
                    h_hbm.at[colbuf.at[j], pl.ds(coff, HALF)], gbuf, semg)

            def gwait(gbuf, semg):
                pltpu.make_async_copy(
                    h_hbm.at[colbuf.at[0], pl.ds(coff, HALF)], gbuf,
                    semg).wait()

            stage(0, rowidx0)
            gather(0, gbuf0, semg0)

            def pair(p, pcarry):
                j0 = 2 * p

                @pl.when(p > 0)
                def _():
                    pltpu.make_async_copy(gbuf1, acc.at[rowidx1],
                                          sems1).wait()

                stage(j0 + 1, rowidx1)
                gather(j0 + 1, gbuf1, semg1)
                gwait(gbuf0, semg0)
                pltpu.async_copy(gbuf0, acc.at[rowidx0], sems0, add=True)
                pltpu.make_async_copy(gbuf0, acc.at[rowidx0], sems0).wait()
                stage(j0 + 2, rowidx0)
                gather(j0 + 2, gbuf0, semg0)
                gwait(gbuf1, semg1)
                pltpu.async_copy(gbuf1, acc.at[rowidx1], sems1, add=True)
                return pcarry

            lax.fori_loop(0, (SEGCH - 1) // 2, pair, 0)
            pltpu.make_async_copy(gbuf1, acc.at[rowidx1], sems1).wait()
            gwait(gbuf0, semg0)
            pltpu.sync_copy(gbuf0, acc.at[rowidx0], add=True)
            return carry

        lax.fori_loop(0, NSEG, seg_body, 0)

        pltpu.sync_copy(degbuf, degacc.at[degids], add=True)
        plsc.subcore_barrier()

        r0 = s * ROWS_PER_SUB
        pltpu.sync_copy(acc.at[pl.ds(r0, ROWS_PER_SUB), :],
                        out_hbm.at[c, pl.ds(r0, ROWS_PER_SUB), :])

        @pl.when(s == NSUB - 1)
        def _():
            t0 = NSUB * ROWS_PER_SUB
            pltpu.sync_copy(acc.at[pl.ds(t0, N_NODE - t0), :],
                            out_hbm.at[c, pl.ds(t0, N_NODE - t0), :])

        @pl.when(s == 0)
        def _():
            pltpu.sync_copy(degacc, deg_hbm.at[c])

    return k(h, rows_r, cols_r)


def _tc_finish(a0, a1, deg_col, w0, w1):
    BR = 400
    grid = (N_NODE // BR,)

    def body(a0_ref, a1_ref, deg_ref, w0_ref, w1_ref, o_ref):
        x0 = a0_ref[...]
        x1 = a1_ref[...]
        r = 1.0 / jnp.maximum(deg_ref[...], 1.0)
        acc = jnp.dot(x0, w0_ref[...], preferred_element_type=jnp.float32)
        acc += jnp.dot(x1, w1_ref[...], preferred_element_type=jnp.float32)
        o_ref[...] = acc * r

    return pl.pallas_call(
        body,
        grid=grid,
        in_specs=[
            pl.BlockSpec((BR, HALF), lambda i: (i, 0)),
            pl.BlockSpec((BR, HALF), lambda i: (i, 0)),
            pl.BlockSpec((BR, 1), lambda i: (i, 0)),
            pl.BlockSpec((HALF, D_IN), lambda i: (0, 0)),
            pl.BlockSpec((HALF, D_IN), lambda i: (0, 0)),
        ],
        out_specs=pl.BlockSpec((BR, D_IN), lambda i: (i, 0)),
        out_shape=jax.ShapeDtypeStruct((N_NODE, N_HEAD * D_OUT), jnp.float32),
    )(a0, a1, deg_col, w0, w1)


def kernel(h, edge_idx, W):
    n, d_in = h.shape
    rows_r = edge_idx[0].reshape(NSUB, NSEG, SEGCH, CHUNK)
    cols_r = edge_idx[1].reshape(NSUB, NSEG, SEGCH, CHUNK)

    agg01, deg01 = _sc_aggregate(h, rows_r, cols_r)
    deg_col = (deg01[0] + deg01[1]).reshape(-1)[:n].reshape(n, 1)

    wcat = jnp.transpose(W, (1, 0, 2)).reshape(d_in, N_HEAD * D_OUT)

    return _tc_finish(agg01[0], agg01[1], deg_col, wcat[:HALF], wcat[HALF:])


# speedup vs baseline: 25.6310x; 1.0530x over previous
import functools

import jax
import jax.numpy as jnp
from jax import lax
from jax.experimental import pallas as pl
from jax.experimental.pallas import tpu as pltpu
from jax.experimental.pallas import tpu_sc as plsc

N_NODE = 10000
N_EDGE = 160000
D_IN = 256
D_OUT = 64
N_HEAD = 4

HALF = 128
NSUB = 16
EDGES_PER_SUB = N_EDGE // NSUB
CHUNK = 80
NCHUNK = EDGES_PER_SUB // CHUNK
NSEG = 5
SEGCH = NCHUNK // NSEG
ROWS_PER_SUB = 624
ZROWS = 8
DEGR = 80


def _sc_aggregate(h, rows_r, cols_r):
    mesh = plsc.VectorSubcoreMesh(core_axis_name="c", subcore_axis_name="s")

    @functools.partial(
        pl.kernel,
        out_type=(
            jax.ShapeDtypeStruct((2, N_NODE, HALF), jnp.float32),
            jax.ShapeDtypeStruct((2, DEGR, 128), jnp.float32),
        ),
        mesh=mesh,
        compiler_params=pltpu.CompilerParams(needs_layout_passes=False),
        scratch_types=[
            pltpu.VMEM((SEGCH, CHUNK), jnp.int32),
            pltpu.VMEM((SEGCH, CHUNK), jnp.int32),
            pltpu.VMEM((CHUNK,), jnp.int32),
            pltpu.VMEM((CHUNK,), jnp.int32),
            pltpu.VMEM((CHUNK,), jnp.int32),
            pltpu.VMEM((CHUNK, HALF), jnp.float32),
            pltpu.VMEM((CHUNK, HALF), jnp.float32),
            pltpu.VMEM((CHUNK, HALF), jnp.float32),
            pltpu.VMEM((DEGR, 128), jnp.float32),
            pltpu.VMEM((DEGR,), jnp.int32),
            pltpu.VMEM_SHARED((N_NODE, HALF), jnp.float32),
            pltpu.VMEM_SHARED((DEGR, 128), jnp.float32),
            pltpu.SemaphoreType.DMA,
            pltpu.SemaphoreType.DMA,
            pltpu.SemaphoreType.DMA,
            pltpu.SemaphoreType.DMA,
            pltpu.SemaphoreType.DMA,
            pltpu.SemaphoreType.DMA,
        ],
    )
    def k(h_hbm, rows_hbm, cols_hbm, out_hbm, deg_hbm,
          rowbuf, colbuf, rowidx0, rowidx1, rowidx2,
          gbuf0, gbuf1, gbuf2, degbuf, degids,
          acc, degacc, semg0, semg1, semg2, sems0, sems1, sems2):
        c = lax.axis_index("c")
        s = lax.axis_index("s")

        zero16 = jnp.zeros((16,), jnp.float32)

        def gz(i, carry):
            for kk in range(HALF // 16):
                gbuf0[i, pl.ds(kk * 16, 16)] = zero16
            return carry

        lax.fori_loop(0, CHUNK, gz, 0)

        def dz(i, carry):
            for kk in range(128 // 16):
                degbuf[i, pl.ds(kk * 16, 16)] = zero16
            return carry

        lax.fori_loop(0, DEGR, dz, 0)
        for kk in range(DEGR // 16):
            degids[pl.ds(kk * 16, 16)] = (
                lax.iota(jnp.int32, 16) + (kk * 16))

        ZTAIL = ROWS_PER_SUB - 7 * CHUNK

        def zbody(i, carry):
            pltpu.async_copy(
                gbuf0, acc.at[pl.ds(s * ROWS_PER_SUB + i * CHUNK, CHUNK), :],
                semg0)
            return carry

        lax.fori_loop(0, 7, zbody, 0)
        pltpu.async_copy(
            gbuf0.at[pl.ds(0, ZTAIL)],
            acc.at[pl.ds(s * ROWS_PER_SUB + 7 * CHUNK, ZTAIL), :], semg0)

        @pl.when(s == NSUB - 1)
        def _():
            pltpu.async_copy(
                gbuf0.at[pl.ds(0, 16)],
                acc.at[pl.ds(NSUB * ROWS_PER_SUB, 16), :], semg1)

        @pl.when(s == 0)
        def _():
            pltpu.async_copy(gbuf0, degacc, semg1)

        def zdrain(i, carry):
            pltpu.make_async_copy(
                gbuf0, acc.at[pl.ds(s * ROWS_PER_SUB, CHUNK), :],
                semg0).wait()
            return carry

        lax.fori_loop(0, 7, zdrain, 0)
        pltpu.make_async_copy(
            gbuf0.at[pl.ds(0, ZTAIL)],
            acc.at[pl.ds(s * ROWS_PER_SUB + 7 * CHUNK, ZTAIL), :],
            semg0).wait()

        @pl.when(s == NSUB - 1)
        def _():
            pltpu.make_async_copy(
                gbuf0.at[pl.ds(0, 16)],
                acc.at[pl.ds(NSUB * ROWS_PER_SUB, 16), :], semg1).wait()

        @pl.when(s == 0)
        def _():
            pltpu.make_async_copy(gbuf0, degacc, semg1).wait()

        plsc.subcore_barrier()

        coff = pl.multiple_of(c * HALF, 128)

        def seg_body(g, carry):
            pltpu.sync_copy(rows_hbm.at[s, g], rowbuf)
            pltpu.sync_copy(cols_hbm.at[s, g], colbuf)

            def dbody(j, dcarry):
                for kk in range(CHUNK // 16):
                    v = rowbuf[j, pl.ds(kk * 16, 16)]
                    cnt, last = plsc.scan_count(v)
                    plsc.addupdate_scatter(
                        degbuf, [lax.shift_right_logical(v, 7),
                                 lax.bitwise_and(v, 127)],
                        cnt.astype(jnp.float32), mask=last)
                return dcarry

            lax.fori_loop(c * 12, 12 + c * 13, dbody, 0)

            GB = (gbuf0, gbuf1, gbuf2)
            RX = (rowidx0, rowidx1, rowidx2)
            SG = (semg0, semg1, semg2)
            SS = (sems0, sems1, sems2)

            def stage(j, b):
                for kk in range(CHUNK // 16):
                    sl = pl.ds(kk * 16, 16)
                    RX[b][sl] = rowbuf[j, sl]

            def gather(j, b):
                pltpu.async_copy(
                    h_hbm.at[colbuf.at[j], pl.ds(coff, HALF)], GB[b], SG[b])

            def gwait(b):
                pltpu.make_async_copy(
                    h_hbm.at[colbuf.at[0], pl.ds(coff, HALF)], GB[b],
                    SG[b]).wait()

            def scat(b):
                pltpu.async_copy(GB[b], acc.at[RX[b]], SS[b], add=True)

            def swait(b):
                pltpu.make_async_copy(GB[b], acc.at[RX[b]], SS[b]).wait()

            stage(0, 0); gather(0, 0)
            stage(1, 1); gather(1, 1)
            stage(2, 2); gather(2, 2)
            gwait(0); scat(0)
            swait(0); stage(3, 0); gather(3, 0)
            gwait(1); scat(1)

            def ring(q, rcarry):
                j0 = 2 + 3 * q
                for b2 in range(3):
                    j = j0 + b2
                    a = (1 + b2) % 3
                    b = (2 + b2) % 3
                    swait(a)
                    stage(j + 2, a)
                    gather(j + 2, a)
                    gwait(b)
                    scat(b)
                return rcarry

            lax.fori_loop(0, 7, ring, 0)
            swait(1)
            gwait(2); scat(2)
            swait(2)
            gwait(0); scat(0)
            swait(0)
            return carry

        lax.fori_loop(0, NSEG, seg_body, 0)

        pltpu.sync_copy(degbuf, degacc.at[degids], add=True)
        plsc.subcore_barrier()

        r0 = s * ROWS_PER_SUB
        pltpu.sync_copy(acc.at[pl.ds(r0, ROWS_PER_SUB), :],
                        out_hbm.at[c, pl.ds(r0, ROWS_PER_SUB), :])

        @pl.when(s == NSUB - 1)
        def _():
            t0 = NSUB * ROWS_PER_SUB
            pltpu.sync_copy(acc.at[pl.ds(t0, N_NODE - t0), :],
                            out_hbm.at[c, pl.ds(t0, N_NODE - t0), :])

        @pl.when(s == 0)
        def _():
            pltpu.sync_copy(degacc, deg_hbm.at[c])

    return k(h, rows_r, cols_r)


def _tc_finish(a0, a1, deg_col, w0, w1):
    BR = 400
    grid = (N_NODE // BR,)

    def body(a0_ref, a1_ref, deg_ref, w0_ref, w1_ref, o_ref):
        x0 = a0_ref[...]
        x1 = a1_ref[...]
        r = 1.0 / jnp.maximum(deg_ref[...], 1.0)
        acc = jnp.dot(x0, w0_ref[...], preferred_element_type=jnp.float32)
        acc += jnp.dot(x1, w1_ref[...], preferred_element_type=jnp.float32)
        o_ref[...] = acc * r

    return pl.pallas_call(
        body,
        grid=grid,
        in_specs=[
            pl.BlockSpec((BR, HALF), lambda i: (i, 0)),
            pl.BlockSpec((BR, HALF), lambda i: (i, 0)),
            pl.BlockSpec((BR, 1), lambda i: (i, 0)),
            pl.BlockSpec((HALF, D_IN), lambda i: (0, 0)),
            pl.BlockSpec((HALF, D_IN), lambda i: (0, 0)),
        ],
        out_specs=pl.BlockSpec((BR, D_IN), lambda i: (i, 0)),
        out_shape=jax.ShapeDtypeStruct((N_NODE, N_HEAD * D_OUT), jnp.float32),
    )(a0, a1, deg_col, w0, w1)


def kernel(h, edge_idx, W):
    n, d_in = h.shape
    rows_r = edge_idx[0].reshape(NSUB, NSEG, SEGCH, CHUNK)
    cols_r = edge_idx[1].reshape(NSUB, NSEG, SEGCH, CHUNK)

    agg01, deg01 = _sc_aggregate(h, rows_r, cols_r)
    deg_col = (deg01[0] + deg01[1]).reshape(-1)[:n].reshape(n, 1)

    wcat = jnp.transpose(W, (1, 0, 2)).reshape(d_in, N_HEAD * D_OUT)

    return _tc_finish(agg01[0], agg01[1], deg_col, wcat[:HALF], wcat[HALF:])


# speedup vs baseline: 28.0216x; 1.0933x over previous
import functools

import jax
import jax.numpy as jnp
from jax import lax
from jax.experimental import pallas as pl
from jax.experimental.pallas import tpu as pltpu
from jax.experimental.pallas import tpu_sc as plsc

N_NODE = 10000
N_EDGE = 160000
D_IN = 256
D_OUT = 64
N_HEAD = 4

HALF = 128
NSUB = 16
EDGES_PER_SUB = N_EDGE // NSUB
CHUNK = 80
NCHUNK = EDGES_PER_SUB // CHUNK
NSEG = 5
SEGCH = NCHUNK // NSEG
ROWS_PER_SUB = 624
ZROWS = 8
DEGR = 80


def _sc_aggregate(h, rows_r, cols_r):
    mesh = plsc.VectorSubcoreMesh(core_axis_name="c", subcore_axis_name="s")

    @functools.partial(
        pl.kernel,
        out_type=(
            jax.ShapeDtypeStruct((2, N_NODE, HALF), jnp.float32),
            jax.ShapeDtypeStruct((2, DEGR, 128), jnp.float32),
        ),
        mesh=mesh,
        compiler_params=pltpu.CompilerParams(needs_layout_passes=False),
        scratch_types=[
            pltpu.VMEM((SEGCH, CHUNK), jnp.int32),
            pltpu.VMEM((SEGCH, CHUNK), jnp.int32),
            pltpu.VMEM((CHUNK,), jnp.int32),
            pltpu.VMEM((CHUNK,), jnp.int32),
            pltpu.VMEM((CHUNK,), jnp.int32),
            pltpu.VMEM((CHUNK, HALF), jnp.float32),
            pltpu.VMEM((CHUNK, HALF), jnp.float32),
            pltpu.VMEM((CHUNK, HALF), jnp.float32),
            pltpu.VMEM((DEGR, 128), jnp.float32),
            pltpu.VMEM((DEGR,), jnp.int32),
            pltpu.VMEM_SHARED((N_NODE, HALF), jnp.float32),
            pltpu.VMEM_SHARED((DEGR, 128), jnp.float32),
            pltpu.SemaphoreType.DMA,
            pltpu.SemaphoreType.DMA,
            pltpu.SemaphoreType.DMA,
            pltpu.SemaphoreType.DMA,
            pltpu.SemaphoreType.DMA,
            pltpu.SemaphoreType.DMA,
        ],
    )
    def k(h_hbm, rows_hbm, cols_hbm, out_hbm, deg_hbm,
          rowbuf, colbuf, rowidx0, rowidx1, rowidx2,
          gbuf0, gbuf1, gbuf2, degbuf, degids,
          acc, degacc, semg0, semg1, semg2, sems0, sems1, sems2):
        c = lax.axis_index("c")
        s = lax.axis_index("s")

        zero16 = jnp.zeros((16,), jnp.float32)

        def gz(i, carry):
            for kk in range(HALF // 16):
                gbuf0[i, pl.ds(kk * 16, 16)] = zero16
            return carry

        lax.fori_loop(0, CHUNK, gz, 0)

        def dz(i, carry):
            for kk in range(128 // 16):
                degbuf[i, pl.ds(kk * 16, 16)] = zero16
            return carry

        lax.fori_loop(0, DEGR, dz, 0)
        for kk in range(DEGR // 16):
            degids[pl.ds(kk * 16, 16)] = (
                lax.iota(jnp.int32, 16) + (kk * 16))

        ZTAIL = ROWS_PER_SUB - 7 * CHUNK

        def zbody(i, carry):
            pltpu.async_copy(
                gbuf0, acc.at[pl.ds(s * ROWS_PER_SUB + i * CHUNK, CHUNK), :],
                semg0)
            return carry

        lax.fori_loop(0, 7, zbody, 0)
        pltpu.async_copy(
            gbuf0.at[pl.ds(0, ZTAIL)],
            acc.at[pl.ds(s * ROWS_PER_SUB + 7 * CHUNK, ZTAIL), :], semg0)

        @pl.when(s == NSUB - 1)
        def _():
            pltpu.async_copy(
                gbuf0.at[pl.ds(0, 16)],
                acc.at[pl.ds(NSUB * ROWS_PER_SUB, 16), :], semg1)

        @pl.when(s == 0)
        def _():
            pltpu.async_copy(gbuf0, degacc, semg1)

        def zdrain(i, carry):
            pltpu.make_async_copy(
                gbuf0, acc.at[pl.ds(s * ROWS_PER_SUB, CHUNK), :],
                semg0).wait()
            return carry

        lax.fori_loop(0, 7, zdrain, 0)
        pltpu.make_async_copy(
            gbuf0.at[pl.ds(0, ZTAIL)],
            acc.at[pl.ds(s * ROWS_PER_SUB + 7 * CHUNK, ZTAIL), :],
            semg0).wait()

        @pl.when(s == NSUB - 1)
        def _():
            pltpu.make_async_copy(
                gbuf0.at[pl.ds(0, 16)],
                acc.at[pl.ds(NSUB * ROWS_PER_SUB, 16), :], semg1).wait()

        @pl.when(s == 0)
        def _():
            pltpu.make_async_copy(gbuf0, degacc, semg1).wait()

        plsc.subcore_barrier()

        coff = pl.multiple_of(c * HALF, 128)

        def seg_body(g, carry):
            pltpu.sync_copy(rows_hbm.at[s, g], rowbuf)
            pltpu.sync_copy(cols_hbm.at[s, g], colbuf)

            def dbody(j, dcarry):
                for kk in range(CHUNK // 16):
                    v = rowbuf[j, pl.ds(kk * 16, 16)]
                    cnt, last = plsc.scan_count(v)
                    plsc.addupdate_scatter(
                        degbuf, [lax.shift_right_logical(v, 7),
                                 lax.bitwise_and(v, 127)],
                        cnt.astype(jnp.float32), mask=last)
                return dcarry

            lax.fori_loop(c * 12, 12 + c * 13, dbody, 0)

            GB = (gbuf0, gbuf1, gbuf2)
            RX = (rowidx0, rowidx1, rowidx2)
            SG = (semg0, semg1, semg2)
            SS = (sems0, sems1, sems2)

            def stage(j, b):
                for kk in range(CHUNK // 16):
                    sl = pl.ds(kk * 16, 16)
                    RX[b][sl] = rowbuf[j, sl]

            def gather(j, b):
                pltpu.async_copy(
                    h_hbm.at[colbuf.at[j], pl.ds(coff, HALF)], GB[b], SG[b])

            def gwait(b):
                pltpu.make_async_copy(
                    h_hbm.at[colbuf.at[0], pl.ds(coff, HALF)], GB[b],
                    SG[b]).wait()

            def scat(b):
                pltpu.async_copy(GB[b], acc.at[RX[b]], SS[b], add=True)

            def swait(b):
                pltpu.make_async_copy(GB[b], acc.at[RX[b]], SS[b]).wait()

            stage(0, 0); gather(0, 0)
            stage(1, 1); gather(1, 1)
            gwait(0); scat(0)
            stage(2, 2); gather(2, 2)
            gwait(1); scat(1)

            def ring(q, rcarry):
                j0 = 2 + 3 * q
                for b2 in range(3):
                    j = j0 + b2
                    a = b2 % 3
                    b = (2 + b2) % 3
                    swait(a)
                    stage(j + 1, a)
                    gather(j + 1, a)
                    gwait(b)
                    scat(b)
                return rcarry

            lax.fori_loop(0, 7, ring, 0)
            swait(0); stage(24, 0); gather(24, 0)
            gwait(2); scat(2)
            swait(1)
            gwait(0); scat(0)
            swait(2)
            swait(0)
            return carry

        lax.fori_loop(0, NSEG, seg_body, 0)

        pltpu.sync_copy(degbuf, degacc.at[degids], add=True)
        plsc.subcore_barrier()

        r0 = s * ROWS_PER_SUB
        pltpu.sync_copy(acc.at[pl.ds(r0, ROWS_PER_SUB), :],
                        out_hbm.at[c, pl.ds(r0, ROWS_PER_SUB), :])

        @pl.when(s == NSUB - 1)
        def _():
            t0 = NSUB * ROWS_PER_SUB
            pltpu.sync_copy(acc.at[pl.ds(t0, N_NODE - t0), :],
                            out_hbm.at[c, pl.ds(t0, N_NODE - t0), :])

        @pl.when(s == 0)
        def _():
            pltpu.sync_copy(degacc, deg_hbm.at[c])

    return k(h, rows_r, cols_r)


def _tc_finish(a0, a1, deg_col, w0, w1):
    BR = 1000
    grid = (N_NODE // BR,)

    def body(a0_ref, a1_ref, deg_ref, w0_ref, w1_ref, o_ref):
        x0 = a0_ref[...]
        x1 = a1_ref[...]
        r = 1.0 / jnp.maximum(deg_ref[...], 1.0)
        acc = jnp.dot(x0, w0_ref[...], preferred_element_type=jnp.float32)
        acc += jnp.dot(x1, w1_ref[...], preferred_element_type=jnp.float32)
        o_ref[...] = acc * r

    return pl.pallas_call(
        body,
        grid=grid,
        in_specs=[
            pl.BlockSpec((BR, HALF), lambda i: (i, 0)),
            pl.BlockSpec((BR, HALF), lambda i: (i, 0)),
            pl.BlockSpec((BR, 1), lambda i: (i, 0)),
            pl.BlockSpec((HALF, D_IN), lambda i: (0, 0)),
            pl.BlockSpec((HALF, D_IN), lambda i: (0, 0)),
        ],
        out_specs=pl.BlockSpec((BR, D_IN), lambda i: (i, 0)),
        out_shape=jax.ShapeDtypeStruct((N_NODE, N_HEAD * D_OUT), jnp.float32),
    )(a0, a1, deg_col, w0, w1)


def kernel(h, edge_idx, W):
    n, d_in = h.shape
    rows_r = edge_idx[0].reshape(NSUB, NSEG, SEGCH, CHUNK)
    cols_r = edge_idx[1].reshape(NSUB, NSEG, SEGCH, CHUNK)

    agg01, deg01 = _sc_aggregate(h, rows_r, cols_r)
    deg_col = (deg01[0] + deg01[1]).reshape(-1)[:n].reshape(n, 1)

    wcat = jnp.transpose(W, (1, 0, 2)).reshape(d_in, N_HEAD * D_OUT)

    return _tc_finish(agg01[0], agg01[1], deg_col, wcat[:HALF], wcat[HALF:])


# speedup vs baseline: 28.0640x; 1.0015x over previous
import functools

import jax
import jax.numpy as jnp
from jax import lax
from jax.experimental import pallas as pl
from jax.experimental.pallas import tpu as pltpu
from jax.experimental.pallas import tpu_sc as plsc

N_NODE = 10000
N_EDGE = 160000
D_IN = 256
D_OUT = 64
N_HEAD = 4

HALF = 128
NSUB = 16
EDGES_PER_SUB = N_EDGE // NSUB
CHUNK = 80
NCHUNK = EDGES_PER_SUB // CHUNK
NSEG = 5
SEGCH = NCHUNK // NSEG
ROWS_PER_SUB = 624
ZROWS = 8
DEGR = 80


def _sc_aggregate(h, rows_r, cols_r):
    mesh = plsc.VectorSubcoreMesh(core_axis_name="c", subcore_axis_name="s")

    @functools.partial(
        pl.kernel,
        out_type=(
            jax.ShapeDtypeStruct((2, N_NODE, HALF), jnp.float32),
            jax.ShapeDtypeStruct((2, DEGR, 128), jnp.float32),
        ),
        mesh=mesh,
        compiler_params=pltpu.CompilerParams(needs_layout_passes=False),
        scratch_types=[
            pltpu.VMEM((SEGCH, CHUNK), jnp.int32),
            pltpu.VMEM((SEGCH, CHUNK), jnp.int32),
            pltpu.VMEM((CHUNK,), jnp.int32),
            pltpu.VMEM((CHUNK,), jnp.int32),
            pltpu.VMEM((CHUNK,), jnp.int32),
            pltpu.VMEM((CHUNK, HALF), jnp.float32),
            pltpu.VMEM((CHUNK, HALF), jnp.float32),
            pltpu.VMEM((CHUNK, HALF), jnp.float32),
            pltpu.VMEM((DEGR, 128), jnp.float32),
            pltpu.VMEM((DEGR,), jnp.int32),
            pltpu.VMEM_SHARED((N_NODE, HALF), jnp.float32),
            pltpu.VMEM_SHARED((DEGR, 128), jnp.float32),
            pltpu.SemaphoreType.DMA,
            pltpu.SemaphoreType.DMA,
            pltpu.SemaphoreType.DMA,
            pltpu.SemaphoreType.DMA,
            pltpu.SemaphoreType.DMA,
            pltpu.SemaphoreType.DMA,
        ],
    )
    def k(h_hbm, rows_hbm, cols_hbm, out_hbm, deg_hbm,
          rowbuf, colbuf, rowidx0, rowidx1, rowidx2,
          gbuf0, gbuf1, gbuf2, degbuf, degids,
          acc, degacc, semg0, semg1, semg2, sems0, sems1, sems2):
        c = lax.axis_index("c")
        s = lax.axis_index("s")

        zero16 = jnp.zeros((16,), jnp.float32)

        def gz(i, carry):
            for kk in range(HALF // 16):
                gbuf0[i, pl.ds(kk * 16, 16)] = zero16
            return carry

        lax.fori_loop(0, CHUNK, gz, 0)

        def dz(i, carry):
            for kk in range(128 // 16):
                degbuf[i, pl.ds(kk * 16, 16)] = zero16
            return carry

        lax.fori_loop(0, DEGR, dz, 0)
        for kk in range(DEGR // 16):
            degids[pl.ds(kk * 16, 16)] = (
                lax.iota(jnp.int32, 16) + (kk * 16))

        ZTAIL = ROWS_PER_SUB - 7 * CHUNK

        def zbody(i, carry):
            pltpu.async_copy(
                gbuf0, acc.at[pl.ds(s * ROWS_PER_SUB + i * CHUNK, CHUNK), :],
                semg0)
            return carry

        lax.fori_loop(0, 7, zbody, 0)
        pltpu.async_copy(
            gbuf0.at[pl.ds(0, ZTAIL)],
            acc.at[pl.ds(s * ROWS_PER_SUB + 7 * CHUNK, ZTAIL), :], semg0)

        @pl.when(s == NSUB - 1)
        def _():
            pltpu.async_copy(
                gbuf0.at[pl.ds(0, 16)],
                acc.at[pl.ds(NSUB * ROWS_PER_SUB, 16), :], semg1)

        @pl.when(s == 0)
        def _():
            pltpu.async_copy(gbuf0, degacc, semg1)

        def zdrain(i, carry):
            pltpu.make_async_copy(
                gbuf0, acc.at[pl.ds(s * ROWS_PER_SUB, CHUNK), :],
                semg0).wait()
            return carry

        lax.fori_loop(0, 7, zdrain, 0)
        pltpu.make_async_copy(
            gbuf0.at[pl.ds(0, ZTAIL)],
            acc.at[pl.ds(s * ROWS_PER_SUB + 7 * CHUNK, ZTAIL), :],
            semg0).wait()

        @pl.when(s == NSUB - 1)
        def _():
            pltpu.make_async_copy(
                gbuf0.at[pl.ds(0, 16)],
                acc.at[pl.ds(NSUB * ROWS_PER_SUB, 16), :], semg1).wait()

        @pl.when(s == 0)
        def _():
            pltpu.make_async_copy(gbuf0, degacc, semg1).wait()

        plsc.subcore_barrier()

        coff = pl.multiple_of(c * HALF, 128)

        def seg_body(g, carry):
            pltpu.sync_copy(rows_hbm.at[s, g], rowbuf)
            pltpu.sync_copy(cols_hbm.at[s, g], colbuf)

            def dbody(j, dcarry):
                for kk in range(CHUNK // 16):
                    v = rowbuf[j, pl.ds(kk * 16, 16)]
                    cnt, last = plsc.scan_count(v)
                    plsc.addupdate_scatter(
                        degbuf, [lax.shift_right_logical(v, 7),
                                 lax.bitwise_and(v, 127)],
                        cnt.astype(jnp.float32), mask=last)
                return dcarry

            lax.fori_loop(c * 12, 12 + c * 13, dbody, 0)

            GB = (gbuf0, gbuf1, gbuf2)
            RX = (rowidx0, rowidx1, rowidx2)
            SG = (semg0, semg1, semg2)
            SS = (sems0, sems1, sems2)

            def stage(j, b):
                for kk in range(CHUNK // 16):
                    sl = pl.ds(kk * 16, 16)
                    RX[b][sl] = rowbuf[j, sl]

            def gather(j, b):
                pltpu.async_copy(
                    h_hbm.at[colbuf.at[j], pl.ds(coff, HALF)], GB[b], SG[b])

            def gwait(b):
                pltpu.make_async_copy(
                    h_hbm.at[colbuf.at[0], pl.ds(coff, HALF)], GB[b],
                    SG[b]).wait()

            def scat(b):
                pltpu.async_copy(GB[b], acc.at[RX[b]], SS[b], add=True)

            def swait(b):
                pltpu.make_async_copy(GB[b], acc.at[RX[b]], SS[b]).wait()

            gather(0, 0); stage(0, 0)
            gather(1, 1); stage(1, 1)
            gwait(0); scat(0)
            gather(2, 2); stage(2, 2)
            gwait(1); scat(1)

            def ring(q, rcarry):
                j0 = 2 + 3 * q
                for b2 in range(3):
                    j = j0 + b2
                    a = b2 % 3
                    b = (2 + b2) % 3
                    swait(a)
                    gather(j + 1, a)
                    stage(j + 1, a)
                    gwait(b)
                    scat(b)
                return rcarry

            lax.fori_loop(0, 7, ring, 0)
            swait(0); gather(24, 0); stage(24, 0)
            gwait(2); scat(2)
            swait(1)
            gwait(0); scat(0)
            swait(2)
            swait(0)
            return carry

        lax.fori_loop(0, NSEG, seg_body, 0)

        pltpu.sync_copy(degbuf, degacc.at[degids], add=True)
        plsc.subcore_barrier()

        r0 = s * ROWS_PER_SUB
        pltpu.sync_copy(acc.at[pl.ds(r0, ROWS_PER_SUB), :],
                        out_hbm.at[c, pl.ds(r0, ROWS_PER_SUB), :])

        @pl.when(s == NSUB - 1)
        def _():
            t0 = NSUB * ROWS_PER_SUB
            pltpu.sync_copy(acc.at[pl.ds(t0, N_NODE - t0), :],
                            out_hbm.at[c, pl.ds(t0, N_NODE - t0), :])

        @pl.when(s == 0)
        def _():
            pltpu.sync_copy(degacc, deg_hbm.at[c])

    return k(h, rows_r, cols_r)


def _tc_finish(a0, a1, deg_col, w0, w1):
    BR = 1000
    grid = (N_NODE // BR,)

    def body(a0_ref, a1_ref, deg_ref, w0_ref, w1_ref, o_ref):
        x0 = a0_ref[...]
        x1 = a1_ref[...]
        r = 1.0 / jnp.maximum(deg_ref[...], 1.0)
        acc = jnp.dot(x0, w0_ref[...], preferred_element_type=jnp.float32)
        acc += jnp.dot(x1, w1_ref[...], preferred_element_type=jnp.float32)
        o_ref[...] = acc * r

    return pl.pallas_call(
        body,
        grid=grid,
        in_specs=[
            pl.BlockSpec((BR, HALF), lambda i: (i, 0)),
            pl.BlockSpec((BR, HALF), lambda i: (i, 0)),
            pl.BlockSpec((BR, 1), lambda i: (i, 0)),
            pl.BlockSpec((HALF, D_IN), lambda i: (0, 0)),
            pl.BlockSpec((HALF, D_IN), lambda i: (0, 0)),
        ],
        out_specs=pl.BlockSpec((BR, D_IN), lambda i: (i, 0)),
        out_shape=jax.ShapeDtypeStruct((N_NODE, N_HEAD * D_OUT), jnp.float32),
    )(a0, a1, deg_col, w0, w1)


def kernel(h, edge_idx, W):
    n, d_in = h.shape
    rows_r = edge_idx[0].reshape(NSUB, NSEG, SEGCH, CHUNK)
    cols_r = edge_idx[1].reshape(NSUB, NSEG, SEGCH, CHUNK)

    agg01, deg01 = _sc_aggregate(h, rows_r, cols_r)
    deg_col = (deg01[0] + deg01[1]).reshape(-1)[:n].reshape(n, 1)

    wcat = jnp.transpose(W, (1, 0, 2)).reshape(d_in, N_HEAD * D_OUT)

    return _tc_finish(agg01[0], agg01[1], deg_col, wcat[:HALF], wcat[HALF:])
